# baseline (speedup 1.0000x reference)
import jax
import jax.numpy as jnp
from jax.experimental import pallas as pl


def _largest_divisor_leq(n, target):
    for b in range(min(target, n) // 8 * 8, 0, -8):
        if n % b == 0:
            return b
    return n


def _support_body(x_ref, w_ref, sup_ref):
    acc = jnp.dot(x_ref[0], w_ref[...], preferred_element_type=jnp.float32)
    sup_ref[...] = acc.astype(jnp.bfloat16)


def _spmm_body(adj_ref, sup_ref, out_ref):
    a = adj_ref[...].astype(jnp.bfloat16)
    acc = jnp.dot(a, sup_ref[...], preferred_element_type=jnp.float32)
    acc = jnp.maximum(acc, 0.0)
    k_slices, d_out = out_ref.shape[0], out_ref.shape[2]
    for k in range(k_slices):
        out_ref[k] = acc[:, k * d_out:(k + 1) * d_out]


def kernel(input, adj, W):
    K, N, D_in = input.shape
    D_out = W.shape[1]

    bn = _largest_divisor_leq(N, 2000)
    sup = pl.pallas_call(
        _support_body,
        grid=(K, N // bn),
        in_specs=[
            pl.BlockSpec((1, bn, D_in), lambda k, i: (k, i, 0)),
            pl.BlockSpec((D_in, D_out), lambda k, i: (0, 0)),
        ],
        out_specs=pl.BlockSpec((bn, D_out), lambda k, i: (i, k)),
        out_shape=jax.ShapeDtypeStruct((N, K * D_out), jnp.bfloat16),
    )(input, W)

    bi = _largest_divisor_leq(N, 200)
    out = pl.pallas_call(
        _spmm_body,
        grid=(N // bi,),
        in_specs=[
            pl.BlockSpec((bi, N), lambda i: (i, 0)),
            pl.BlockSpec((N, K * D_out), lambda i: (0, 0)),
        ],
        out_specs=pl.BlockSpec((K, bi, D_out), lambda i: (0, i, 0)),
        out_shape=jax.ShapeDtypeStruct((K, N, D_out), jnp.float32),
    )(adj, sup)
    return out


# speedup vs baseline: 1.0879x; 1.0879x over previous
import jax
import jax.numpy as jnp
from jax.experimental import pallas as pl
from jax.experimental.pallas import tpu as pltpu


def _largest_divisor_leq(n, target):
    for b in range(min(target, n) // 8 * 8, 0, -8):
        if n % b == 0:
            return b
    return n


def _support_body(x_ref, w_ref, sup_ref):
    acc = jnp.dot(x_ref[0], w_ref[...], preferred_element_type=jnp.float32)
    sup_ref[...] = acc.astype(jnp.bfloat16)


def _spmm_body(adj0_ref, adj1_ref, sup_ref, out_ref):
    k_slices, d_out = out_ref.shape[0], out_ref.shape[2]
    bh = adj0_ref.shape[0]
    sup = sup_ref[...]
    for half, adj_ref in enumerate((adj0_ref, adj1_ref)):
        a = adj_ref[...].astype(jnp.bfloat16)
        acc = jnp.dot(a, sup, preferred_element_type=jnp.float32)
        acc = jnp.maximum(acc, 0.0)
        for k in range(k_slices):
            out_ref[k, half * bh:(half + 1) * bh] = (
                acc[:, k * d_out:(k + 1) * d_out])


def kernel(input, adj, W):
    K, N, D_in = input.shape
    D_out = W.shape[1]

    bn = _largest_divisor_leq(N, 2000)
    sup = pl.pallas_call(
        _support_body,
        grid=(K, N // bn),
        in_specs=[
            pl.BlockSpec((1, bn, D_in), lambda k, i: (k, i, 0)),
            pl.BlockSpec((D_in, D_out), lambda k, i: (0, 0)),
        ],
        out_specs=pl.BlockSpec((bn, D_out), lambda k, i: (i, k)),
        out_shape=jax.ShapeDtypeStruct((N, K * D_out), jnp.bfloat16),
    )(input, W)

    bi = _largest_divisor_leq(N, 400)
    bh = bi // 2
    out = pl.pallas_call(
        _spmm_body,
        grid=(N // bi,),
        in_specs=[
            pl.BlockSpec((bh, N), lambda i: (2 * i, 0)),
            pl.BlockSpec((bh, N), lambda i: (2 * i + 1, 0)),
            pl.BlockSpec((N, K * D_out), lambda i: (0, 0)),
        ],
        out_specs=pl.BlockSpec((K, bi, D_out), lambda i: (0, i, 0)),
        out_shape=jax.ShapeDtypeStruct((K, N, D_out), jnp.float32),
    )(adj, adj, sup)
    return out


# speedup vs baseline: 1.2187x; 1.1202x over previous
import jax
import jax.numpy as jnp
from jax.experimental import pallas as pl
from jax.experimental.pallas import tpu as pltpu


def _largest_divisor_leq(n, target):
    for b in range(min(target, n) // 8 * 8, 0, -8):
        if n % b == 0:
            return b
    return n


def kernel(input, adj, W):
    K, N, D_in = input.shape
    D_out = W.shape[1]

    bs = _largest_divisor_leq(N, 2000)
    S = N // bs
    bi = _largest_divisor_leq(N, 400)
    G = N // bi

    def body(x_ref, adj_ref, w_ref, out_ref, sup_ref):
        i = pl.program_id(0)

        @pl.when(i < S)
        def _support():
            w = w_ref[...]
            for k in range(K):
                acc = jnp.dot(x_ref[k], w, preferred_element_type=jnp.float32)
                sup_ref[pl.ds(i * bs, bs), k * D_out:(k + 1) * D_out] = (
                    acc.astype(jnp.bfloat16))

        @pl.when(i >= S)
        def _spmm():
            a = adj_ref[...].astype(jnp.bfloat16)
            acc = jnp.dot(a, sup_ref[...], preferred_element_type=jnp.float32)
            acc = jnp.maximum(acc, 0.0)
            for k in range(K):
                out_ref[k] = acc[:, k * D_out:(k + 1) * D_out]

    out = pl.pallas_call(
        body,
        grid=(S + G,),
        in_specs=[
            pl.BlockSpec((K, bs, D_in), lambda i: (0, jnp.minimum(i, S - 1), 0)),
            pl.BlockSpec((bi, N), lambda i: (jnp.maximum(i - S, 0), 0)),
            pl.BlockSpec((D_in, D_out), lambda i: (0, 0)),
        ],
        out_specs=pl.BlockSpec(
            (K, bi, D_out), lambda i: (0, jnp.maximum(i - S, 0), 0)),
        out_shape=jax.ShapeDtypeStruct((K, N, D_out), jnp.float32),
        scratch_shapes=[pltpu.VMEM((N, K * D_out), jnp.bfloat16)],
    )(input, adj, W)
    return out
